# baseline (speedup 1.0000x reference)
import functools

import jax
import jax.numpy as jnp
from jax import lax
from jax.experimental import pallas as pl
from jax.experimental.pallas import tpu as pltpu
from jax.experimental.pallas import tpu_sc as plsc

T_SIZE = 8192
D = 128
NC = 2
NS = 16
NW = NC * NS
ROWS_PER_W = T_SIZE // NW
CHUNK = 64
NCH = ROWS_PER_W // CHUNK


def _gather_body(tbl_hbm, idx_hbm, out_hbm, idx_v, rows_v, gsem, wsem):
    wid = lax.axis_index("s") * NC + lax.axis_index("c")
    base = wid * NCH
    pltpu.sync_copy(idx_hbm.at[pl.ds(base, NCH)], idx_v)
    gathers = [
        pltpu.async_copy(tbl_hbm.at[idx_v.at[j]], rows_v.at[j], gsem)
        for j in range(NCH)
    ]
    for g in gathers:
        g.wait()
    pltpu.async_copy(rows_v, out_hbm.at[pl.ds(base, NCH)], wsem).wait()


@jax.jit
def kernel(trpe, TDist):
    idx = TDist.reshape(NW * NCH, CHUNK)
    run = pl.kernel(
        _gather_body,
        out_type=jax.ShapeDtypeStruct((NW * NCH, CHUNK, D), jnp.float32),
        mesh=plsc.VectorSubcoreMesh(
            core_axis_name="c", subcore_axis_name="s",
            num_cores=NC, num_subcores=NS,
        ),
        scratch_types=[
            pltpu.VMEM((NCH, CHUNK), jnp.int32),
            pltpu.VMEM((NCH, CHUNK, D), jnp.float32),
            pltpu.SemaphoreType.DMA,
            pltpu.SemaphoreType.DMA,
        ],
    )
    out = run(trpe, idx)
    return out.reshape(T_SIZE, 1, D)


# speedup vs baseline: 1.0205x; 1.0205x over previous
import functools

import jax
import jax.numpy as jnp
from jax import lax
from jax.experimental import pallas as pl
from jax.experimental.pallas import tpu as pltpu
from jax.experimental.pallas import tpu_sc as plsc

T_SIZE = 8192
D = 128
NC = 2
NS = 16
NW = NC * NS
ROWS_PER_W = T_SIZE // NW
CHUNK = 128
NCH = ROWS_PER_W // CHUNK


def _gather_body(tbl_hbm, idx_hbm, out_hbm, idx_v, rows_v, gsem, wsem):
    wid = lax.axis_index("s") * NC + lax.axis_index("c")
    base = wid * NCH
    pltpu.sync_copy(idx_hbm.at[pl.ds(base, NCH)], idx_v)
    gathers = [
        pltpu.async_copy(tbl_hbm.at[idx_v.at[j]], rows_v.at[j], gsem)
        for j in range(NCH)
    ]
    for g in gathers:
        g.wait()
    pltpu.async_copy(rows_v, out_hbm.at[pl.ds(base, NCH)], wsem).wait()


@jax.jit
def kernel(trpe, TDist):
    idx = TDist.reshape(NW * NCH, CHUNK)
    run = pl.kernel(
        _gather_body,
        out_type=jax.ShapeDtypeStruct((NW * NCH, CHUNK, D), jnp.float32),
        mesh=plsc.VectorSubcoreMesh(
            core_axis_name="c", subcore_axis_name="s",
            num_cores=NC, num_subcores=NS,
        ),
        scratch_types=[
            pltpu.VMEM((NCH, CHUNK), jnp.int32),
            pltpu.VMEM((NCH, CHUNK, D), jnp.float32),
            pltpu.SemaphoreType.DMA,
            pltpu.SemaphoreType.DMA,
        ],
    )
    out = run(trpe, idx)
    return out.reshape(T_SIZE, 1, D)


# speedup vs baseline: 1.0276x; 1.0070x over previous
import jax
import jax.numpy as jnp
from jax import lax
from jax.experimental import pallas as pl
from jax.experimental.pallas import tpu as pltpu
from jax.experimental.pallas import tpu_sc as plsc

T_SIZE = 8192
D = 128
NC = 2
NS = 16
NW = NC * NS
ROWS_PER_W = T_SIZE // NW
CHUNK = 128
NCH = ROWS_PER_W // CHUNK


def _gather_body(tbl_hbm, idx_hbm, out_hbm, idx_v, rows_v, gsem, wsem):
    wid = lax.axis_index("s") * NC + lax.axis_index("c")
    base = wid * NCH
    pltpu.sync_copy(idx_hbm.at[pl.ds(base, NCH)], idx_v)
    gathers = [
        pltpu.async_copy(tbl_hbm.at[idx_v.at[j]], rows_v.at[j], gsem)
        for j in range(NCH)
    ]
    for g in gathers:
        g.wait()
    pltpu.async_copy(rows_v, out_hbm.at[pl.ds(base, NCH)], wsem).wait()


@jax.jit
def kernel(trpe, TDist):
    idx = TDist.reshape(NW * NCH, CHUNK)
    run = pl.kernel(
        _gather_body,
        out_type=jax.ShapeDtypeStruct((NW * NCH, CHUNK, D), jnp.float32),
        mesh=plsc.VectorSubcoreMesh(
            core_axis_name="c", subcore_axis_name="s",
            num_cores=NC, num_subcores=NS,
        ),
        scratch_types=[
            pltpu.VMEM((NCH, CHUNK), jnp.int32),
            pltpu.VMEM((NCH, CHUNK, D), jnp.float32),
            pltpu.SemaphoreType.DMA,
            pltpu.SemaphoreType.DMA,
        ],
    )
    out = run(trpe, idx)
    return out.reshape(T_SIZE, 1, D)
